# baseline (speedup 1.0000x reference)
import functools

import jax
import jax.numpy as jnp
from jax import lax
from jax.experimental import pallas as pl
from jax.experimental.pallas import tpu as pltpu
from jax.experimental.pallas import tpu_sc as plsc

D_MODEL = 1024
D_EMBED = 128
N_CHR = 32
ECC_BITS = 32
K = 32
BATCH = 4
SEQ = 2048
NROWS = BATCH * SEQ

ROWS_PER_BLOCK = 512
SIM_ROWS = 256

NC, NS = 2, 16
NW = NC * NS
ROWS_PER_W = NROWS // NW
RCHUNK = 8
NCHUNKS = ROWS_PER_W // RCHUNK


def _proja_body(x_ref, we_ref, be_ref, emb_ref, embn_ref):
    x = x_ref[...]
    emb = jnp.dot(x, we_ref[...], preferred_element_type=jnp.float32) + be_ref[...]
    emb_ref[...] = emb
    nrm = jnp.sqrt(jnp.sum(emb * emb, axis=-1, keepdims=True)) + 1e-8
    embn_ref[...] = emb / nrm


def _simmat_body(rows_ref, cols_ref, sim_ref):
    i = pl.program_id(1)
    sim = lax.dot_general(rows_ref[0], cols_ref[0], (((1,), (1,)), ((), ())),
                          preferred_element_type=jnp.float32)
    col_iota = lax.broadcasted_iota(jnp.int32, (SIM_ROWS, SEQ), 1)
    row_ids = i * SIM_ROWS + lax.broadcasted_iota(jnp.int32, (SIM_ROWS, SEQ), 0)
    sim_ref[...] = jnp.where(col_iota == row_ids, sim - 1e9, sim)


def _projc_body(x_ref, wd_ref, wc_ref, bc_ref, wp_ref, bp_ref,
                w1_ref, b1_ref, w2_ref, b2_ref,
                met_ref, chr_ref, ecc_ref):
    x = x_ref[...]
    met_ref[...] = jnp.dot(x, wd_ref[...], preferred_element_type=jnp.float32)
    chr_ref[...] = jnp.dot(x, wc_ref[...], preferred_element_type=jnp.float32) + bc_ref[...]
    p = jnp.dot(x, wp_ref[...], preferred_element_type=jnp.float32) + bp_ref[...]
    h = jnp.tanh(jnp.dot(p, w1_ref[...], preferred_element_type=jnp.float32) + b1_ref[...])
    e = jnp.dot(h, w2_ref[...], preferred_element_type=jnp.float32) + b2_ref[...]
    ecc_ref[...] = 1.0 / (1.0 + jnp.exp(-e))


def _merge32(k1, v1, k2, v2, bk1, bv1, bk2, bv2):
    rbk2 = lax.rev(bk2, (0,)); rbv2 = lax.rev(bv2, (0,))
    rbk1 = lax.rev(bk1, (0,)); rbv1 = lax.rev(bv1, (0,))
    m1 = k1 >= rbk2
    t1k = jnp.where(m1, k1, rbk2); t1v = jnp.where(m1, v1, rbv2)
    m2 = k2 >= rbk1
    t2k = jnp.where(m2, k2, rbk1); t2v = jnp.where(m2, v2, rbv1)
    mm = t1k >= t2k
    u1k = jnp.where(mm, t1k, t2k); u1v = jnp.where(mm, t1v, t2v)
    u2k = jnp.where(mm, t2k, t1k); u2v = jnp.where(mm, t2v, t1v)
    s1k, s1v = plsc.sort_key_val(u1k, u1v, descending=True)
    s2k, s2v = plsc.sort_key_val(u2k, u2v, descending=True)
    return s1k, s1v, s2k, s2v


def _sctopk_body(sim_hbm, sc_hbm, ix_hbm,
                 rows_v, key_v, idx_v, keyb_v, idxb_v, rs_v, ri_v):
    wid = lax.axis_index("s") * NC + lax.axis_index("c")
    base = wid * ROWS_PER_W
    iota = lax.iota(jnp.int32, 16)

    def chunk_body(g, _):
        pltpu.sync_copy(sim_hbm.at[pl.ds(base + g * RCHUNK, RCHUNK)], rows_v)

        def row_body(r, _):
            def l01(j, _):
                off = j * 32
                ak = rows_v[r, pl.ds(off, 16)]
                bk = rows_v[r, pl.ds(off + 16, 16)]
                ia = off + iota
                ib = ia + 16
                sak, sav = plsc.sort_key_val(ak, ia, descending=True)
                sbk, sbv = plsc.sort_key_val(bk, ib, descending=True)
                rbk = lax.rev(sbk, (0,)); rbv = lax.rev(sbv, (0,))
                m = sak >= rbk
                u1k = jnp.where(m, sak, rbk); u1v = jnp.where(m, sav, rbv)
                u2k = jnp.where(m, rbk, sak); u2v = jnp.where(m, rbv, sav)
                s1k, s1v = plsc.sort_key_val(u1k, u1v, descending=True)
                s2k, s2v = plsc.sort_key_val(u2k, u2v, descending=True)
                key_v[pl.ds(off, 16)] = s1k
                key_v[pl.ds(off + 16, 16)] = s2k
                idx_v[pl.ds(off, 16)] = s1v
                idx_v[pl.ds(off + 16, 16)] = s2v
                return 0

            lax.fori_loop(0, SEQ // 32, l01, 0, unroll=2)

            nlists = SEQ // 32
            srck, srci, dstk, dsti = key_v, idx_v, keyb_v, idxb_v
            while nlists > 1:
                def lvl(j, _, srck=srck, srci=srci, dstk=dstk, dsti=dsti):
                    a = j * 64
                    r_ = _merge32(
                        srck[pl.ds(a, 16)], srci[pl.ds(a, 16)],
                        srck[pl.ds(a + 16, 16)], srci[pl.ds(a + 16, 16)],
                        srck[pl.ds(a + 32, 16)], srci[pl.ds(a + 32, 16)],
                        srck[pl.ds(a + 48, 16)], srci[pl.ds(a + 48, 16)])
                    o = j * 32
                    dstk[pl.ds(o, 16)] = r_[0]
                    dstk[pl.ds(o + 16, 16)] = r_[2]
                    dsti[pl.ds(o, 16)] = r_[1]
                    dsti[pl.ds(o + 16, 16)] = r_[3]
                    return 0

                lax.fori_loop(0, nlists // 2, lvl, 0,
                              unroll=2 if nlists > 4 else 1)
                srck, srci, dstk, dsti = dstk, dsti, srck, srci
                nlists //= 2

            rr = g * RCHUNK + r
            rs_v[rr, pl.ds(0, 16)] = srck[pl.ds(0, 16)]
            rs_v[rr, pl.ds(16, 16)] = srck[pl.ds(16, 16)]
            ri_v[rr, pl.ds(0, 16)] = srci[pl.ds(0, 16)]
            ri_v[rr, pl.ds(16, 16)] = srci[pl.ds(16, 16)]
            return 0

        lax.fori_loop(0, RCHUNK, row_body, 0)
        return 0

    lax.fori_loop(0, NCHUNKS, chunk_body, 0)
    pltpu.sync_copy(rs_v, sc_hbm.at[pl.ds(base, ROWS_PER_W)])
    pltpu.sync_copy(ri_v, ix_hbm.at[pl.ds(base, ROWS_PER_W)])


def kernel(x, W_embed, b_embed, W_diag, W_chr, b_chr,
           W_ecc_proj, b_ecc_proj, W_e1, b_e1, W_e2, b_e2):
    xf = x.reshape(NROWS, D_MODEL)
    nblk = NROWS // ROWS_PER_BLOCK

    def _full(shape):
        return pl.BlockSpec(shape, lambda i: (0,) * len(shape))

    emb, embn = pl.pallas_call(
        _proja_body,
        grid=(nblk,),
        in_specs=[
            pl.BlockSpec((ROWS_PER_BLOCK, D_MODEL), lambda i: (i, 0)),
            _full((D_MODEL, D_EMBED)), _full((1, D_EMBED)),
        ],
        out_specs=[
            pl.BlockSpec((ROWS_PER_BLOCK, D_EMBED), lambda i: (i, 0)),
            pl.BlockSpec((ROWS_PER_BLOCK, D_EMBED), lambda i: (i, 0)),
        ],
        out_shape=[
            jax.ShapeDtypeStruct((NROWS, D_EMBED), jnp.float32),
            jax.ShapeDtypeStruct((NROWS, D_EMBED), jnp.float32),
        ],
    )(xf, W_embed, b_embed.reshape(1, -1))

    embn3 = embn.reshape(BATCH, SEQ, D_EMBED)
    nrow = SEQ // SIM_ROWS
    sim = pl.pallas_call(
        _simmat_body,
        grid=(BATCH, nrow),
        in_specs=[
            pl.BlockSpec((1, SIM_ROWS, D_EMBED), lambda b, i: (b, i, 0)),
            pl.BlockSpec((1, SEQ, D_EMBED), lambda b, i: (b, 0, 0)),
        ],
        out_specs=pl.BlockSpec((SIM_ROWS, SEQ), lambda b, i: (b * nrow + i, 0)),
        out_shape=jax.ShapeDtypeStruct((NROWS, SEQ), jnp.float32),
    )(embn3, embn3)

    knn_sc, knn_ix = pl.kernel(
        _sctopk_body,
        out_type=[jax.ShapeDtypeStruct((NROWS, K), jnp.float32),
                  jax.ShapeDtypeStruct((NROWS, K), jnp.int32)],
        mesh=plsc.VectorSubcoreMesh(core_axis_name="c", subcore_axis_name="s",
                                    num_cores=NC, num_subcores=NS),
        scratch_types=[
            pltpu.VMEM((RCHUNK, SEQ), jnp.float32),
            pltpu.VMEM((SEQ,), jnp.float32),
            pltpu.VMEM((SEQ,), jnp.int32),
            pltpu.VMEM((SEQ,), jnp.float32),
            pltpu.VMEM((SEQ,), jnp.int32),
            pltpu.VMEM((ROWS_PER_W, K), jnp.float32),
            pltpu.VMEM((ROWS_PER_W, K), jnp.int32),
        ],
        compiler_params=pltpu.CompilerParams(needs_layout_passes=False),
    )(sim)

    met, chrs, ecc = pl.pallas_call(
        _projc_body,
        grid=(nblk,),
        in_specs=[
            pl.BlockSpec((ROWS_PER_BLOCK, D_MODEL), lambda i: (i, 0)),
            _full((D_MODEL, D_MODEL)),
            _full((D_MODEL, N_CHR)), _full((1, N_CHR)),
            _full((D_MODEL, ECC_BITS)), _full((1, ECC_BITS)),
            _full((ECC_BITS, ECC_BITS * 2)), _full((1, ECC_BITS * 2)),
            _full((ECC_BITS * 2, ECC_BITS)), _full((1, ECC_BITS)),
        ],
        out_specs=[
            pl.BlockSpec((ROWS_PER_BLOCK, D_MODEL), lambda i: (i, 0)),
            pl.BlockSpec((ROWS_PER_BLOCK, N_CHR), lambda i: (i, 0)),
            pl.BlockSpec((ROWS_PER_BLOCK, ECC_BITS), lambda i: (i, 0)),
        ],
        out_shape=[
            jax.ShapeDtypeStruct((NROWS, D_MODEL), jnp.float32),
            jax.ShapeDtypeStruct((NROWS, N_CHR), jnp.float32),
            jax.ShapeDtypeStruct((NROWS, ECC_BITS), jnp.float32),
        ],
    )(xf, W_diag, W_chr, b_chr.reshape(1, -1),
      W_ecc_proj, b_ecc_proj.reshape(1, -1), W_e1, b_e1.reshape(1, -1),
      W_e2, b_e2.reshape(1, -1))

    embedding = emb.reshape(BATCH, SEQ, D_EMBED)
    metric = met.reshape(BATCH, SEQ, D_MODEL)
    christoffel = chrs.reshape(BATCH, SEQ, N_CHR)
    ecc_out = ecc.reshape(BATCH, SEQ, ECC_BITS)
    knn_scores = knn_sc.reshape(BATCH, SEQ, K)
    knn_indices = knn_ix.reshape(BATCH, SEQ, K)
    half_k = K // 2
    min_heap = knn_scores[..., :half_k]
    max_heap = -knn_scores[..., half_k:]
    return (embedding, metric, christoffel, knn_scores, knn_indices,
            min_heap, max_heap, ecc_out)


# speedup vs baseline: 1.7631x; 1.7631x over previous
import functools

import jax
import jax.numpy as jnp
from jax import lax
from jax.experimental import pallas as pl
from jax.experimental.pallas import tpu as pltpu
from jax.experimental.pallas import tpu_sc as plsc

D_MODEL = 1024
D_EMBED = 128
N_CHR = 32
ECC_BITS = 32
K = 32
BATCH = 4
SEQ = 2048
NROWS = BATCH * SEQ

ROWS_PER_BLOCK = 512
SIM_ROWS = 256

NC, NS = 2, 16
NW = NC * NS
S_SC = 4352
ROWS_PER_W = S_SC // NW
RCHUNK = 8
NCHUNKS = ROWS_PER_W // RCHUNK


def _proja_body(x_ref, we_ref, be_ref, emb_ref):
    c = D_MODEL // 4
    acc = jnp.dot(x_ref[:, :c], we_ref[:c, :], preferred_element_type=jnp.float32)
    for i in range(1, 4):
        acc = acc + jnp.dot(x_ref[:, i * c:(i + 1) * c], we_ref[i * c:(i + 1) * c, :],
                            preferred_element_type=jnp.float32)
    emb_ref[...] = acc + be_ref[...]


def _simmat_body(rows_ref, cols_ref, sim_ref):
    j = pl.program_id(0)
    sim = lax.dot_general(rows_ref[0], cols_ref[0], (((1,), (1,)), ((), ())),
                          preferred_element_type=jnp.float32)
    col_iota = lax.broadcasted_iota(jnp.int32, (SIM_ROWS, SEQ), 1)
    row_ids = ((j % (SEQ // SIM_ROWS)) * SIM_ROWS
               + lax.broadcasted_iota(jnp.int32, (SIM_ROWS, SEQ), 0))
    sim_ref[...] = jnp.where(col_iota == row_ids, sim - 1e9, sim)


def _tctopk_body(rows_ref, cols_ref, sc_ref, ix_ref):
    j = pl.program_id(0)
    sim = lax.dot_general(rows_ref[0], cols_ref[0], (((1,), (1,)), ((), ())),
                          preferred_element_type=jnp.float32)
    col_iota = lax.broadcasted_iota(jnp.int32, (SIM_ROWS, SEQ), 1)
    row_ids = (((S_SC // SIM_ROWS + j) % (SEQ // SIM_ROWS)) * SIM_ROWS
               + lax.broadcasted_iota(jnp.int32, (SIM_ROWS, SEQ), 0))
    sim = jnp.where(col_iota == row_ids, sim - 1e9, sim)
    scores = []
    idxs = []
    neg = jnp.float32(-jnp.inf)
    for _ in range(K):
        m = jnp.max(sim, axis=1, keepdims=True)
        am = jnp.min(jnp.where(sim == m, col_iota, SEQ), axis=1, keepdims=True)
        scores.append(m)
        idxs.append(am)
        sim = jnp.where(col_iota == am, neg, sim)
    sc_ref[...] = jnp.concatenate(scores, axis=1)
    ix_ref[...] = jnp.concatenate(idxs, axis=1)


def _projc_body(x_ref, wd_ref, wc_ref, bc_ref, wp_ref, bp_ref,
                w1_ref, b1_ref, w2_ref, b2_ref,
                met_ref, chr_ref, ecc_ref):
    x = x_ref[...]
    met_ref[...] = jnp.dot(x, wd_ref[...], preferred_element_type=jnp.float32)
    chr_ref[...] = jnp.dot(x, wc_ref[...], preferred_element_type=jnp.float32) + bc_ref[...]
    p = jnp.dot(x, wp_ref[...], preferred_element_type=jnp.float32) + bp_ref[...]
    h = jnp.tanh(jnp.dot(p, w1_ref[...], preferred_element_type=jnp.float32) + b1_ref[...])
    e = jnp.dot(h, w2_ref[...], preferred_element_type=jnp.float32) + b2_ref[...]
    ecc_ref[...] = 1.0 / (1.0 + jnp.exp(-e))


def _merge32(k1, v1, k2, v2, bk1, bv1, bk2, bv2):
    rbk2 = lax.rev(bk2, (0,)); rbv2 = lax.rev(bv2, (0,))
    rbk1 = lax.rev(bk1, (0,)); rbv1 = lax.rev(bv1, (0,))
    m1 = k1 >= rbk2
    t1k = jnp.where(m1, k1, rbk2); t1v = jnp.where(m1, v1, rbv2)
    m2 = k2 >= rbk1
    t2k = jnp.where(m2, k2, rbk1); t2v = jnp.where(m2, v2, rbv1)
    mm = t1k >= t2k
    u1k = jnp.where(mm, t1k, t2k); u1v = jnp.where(mm, t1v, t2v)
    u2k = jnp.where(mm, t2k, t1k); u2v = jnp.where(mm, t2v, t1v)
    s1k, s1v = plsc.sort_key_val(u1k, u1v, descending=True)
    s2k, s2v = plsc.sort_key_val(u2k, u2v, descending=True)
    return s1k, s1v, s2k, s2v


def _sctopk_body(sim_hbm, sc_hbm, ix_hbm,
                 rows_v, key_v, idx_v, keyb_v, idxb_v, rs_v, ri_v):
    wid = lax.axis_index("s") * NC + lax.axis_index("c")
    base = wid * ROWS_PER_W
    iota = lax.iota(jnp.int32, 16)

    def chunk_body(g, _):
        pltpu.sync_copy(sim_hbm.at[pl.ds(base + g * RCHUNK, RCHUNK)], rows_v)

        def row_body(r, _):
            def l01(j, _):
                off = j * 32
                ak = rows_v[r, pl.ds(off, 16)]
                bk = rows_v[r, pl.ds(off + 16, 16)]
                ia = off + iota
                ib = ia + 16
                sak, sav = plsc.sort_key_val(ak, ia, descending=True)
                sbk, sbv = plsc.sort_key_val(bk, ib, descending=True)
                rbk = lax.rev(sbk, (0,)); rbv = lax.rev(sbv, (0,))
                m = sak >= rbk
                u1k = jnp.where(m, sak, rbk); u1v = jnp.where(m, sav, rbv)
                u2k = jnp.where(m, rbk, sak); u2v = jnp.where(m, rbv, sav)
                s1k, s1v = plsc.sort_key_val(u1k, u1v, descending=True)
                s2k, s2v = plsc.sort_key_val(u2k, u2v, descending=True)
                key_v[pl.ds(off, 16)] = s1k
                key_v[pl.ds(off + 16, 16)] = s2k
                idx_v[pl.ds(off, 16)] = s1v
                idx_v[pl.ds(off + 16, 16)] = s2v
                return 0

            lax.fori_loop(0, SEQ // 32, l01, 0, unroll=2)

            nlists = SEQ // 32
            srck, srci, dstk, dsti = key_v, idx_v, keyb_v, idxb_v
            while nlists > 1:
                def lvl(j, _, srck=srck, srci=srci, dstk=dstk, dsti=dsti):
                    a = j * 64
                    r_ = _merge32(
                        srck[pl.ds(a, 16)], srci[pl.ds(a, 16)],
                        srck[pl.ds(a + 16, 16)], srci[pl.ds(a + 16, 16)],
                        srck[pl.ds(a + 32, 16)], srci[pl.ds(a + 32, 16)],
                        srck[pl.ds(a + 48, 16)], srci[pl.ds(a + 48, 16)])
                    o = j * 32
                    dstk[pl.ds(o, 16)] = r_[0]
                    dstk[pl.ds(o + 16, 16)] = r_[2]
                    dsti[pl.ds(o, 16)] = r_[1]
                    dsti[pl.ds(o + 16, 16)] = r_[3]
                    return 0

                lax.fori_loop(0, nlists // 2, lvl, 0,
                              unroll=2 if nlists > 4 else 1)
                srck, srci, dstk, dsti = dstk, dsti, srck, srci
                nlists //= 2

            rr = g * RCHUNK + r
            rs_v[rr, pl.ds(0, 16)] = srck[pl.ds(0, 16)]
            rs_v[rr, pl.ds(16, 16)] = srck[pl.ds(16, 16)]
            ri_v[rr, pl.ds(0, 16)] = srci[pl.ds(0, 16)]
            ri_v[rr, pl.ds(16, 16)] = srci[pl.ds(16, 16)]
            return 0

        lax.fori_loop(0, RCHUNK, row_body, 0)
        return 0

    lax.fori_loop(0, NCHUNKS, chunk_body, 0)
    pltpu.sync_copy(rs_v, sc_hbm.at[pl.ds(base, ROWS_PER_W)])
    pltpu.sync_copy(ri_v, ix_hbm.at[pl.ds(base, ROWS_PER_W)])


def kernel(x, W_embed, b_embed, W_diag, W_chr, b_chr,
           W_ecc_proj, b_ecc_proj, W_e1, b_e1, W_e2, b_e2):
    xf = x.reshape(NROWS, D_MODEL)
    nblk = NROWS // ROWS_PER_BLOCK

    def _full(shape):
        return pl.BlockSpec(shape, lambda i: (0,) * len(shape))

    emb = pl.pallas_call(
        _proja_body,
        grid=(nblk,),
        in_specs=[
            pl.BlockSpec((ROWS_PER_BLOCK, D_MODEL), lambda i: (i, 0)),
            _full((D_MODEL, D_EMBED)), _full((1, D_EMBED)),
        ],
        out_specs=pl.BlockSpec((ROWS_PER_BLOCK, D_EMBED), lambda i: (i, 0)),
        out_shape=jax.ShapeDtypeStruct((NROWS, D_EMBED), jnp.float32),
    )(xf, W_embed, b_embed.reshape(1, -1))

    embedding = emb.reshape(BATCH, SEQ, D_EMBED)
    embn3 = embedding / (jnp.linalg.norm(embedding, axis=-1, keepdims=True) + 1e-8)
    nrow = SEQ // SIM_ROWS
    nblk_sc = S_SC // SIM_ROWS
    sim = pl.pallas_call(
        _simmat_body,
        grid=(nblk_sc,),
        in_specs=[
            pl.BlockSpec((1, SIM_ROWS, D_EMBED),
                         lambda j: (j // nrow, j % nrow, 0)),
            pl.BlockSpec((1, SEQ, D_EMBED), lambda j: (j // nrow, 0, 0)),
        ],
        out_specs=pl.BlockSpec((SIM_ROWS, SEQ), lambda j: (j, 0)),
        out_shape=jax.ShapeDtypeStruct((S_SC, SEQ), jnp.float32),
    )(embn3, embn3)

    nblk_tc = (NROWS - S_SC) // SIM_ROWS
    off = S_SC // SIM_ROWS
    tc_sc, tc_ix = pl.pallas_call(
        _tctopk_body,
        grid=(nblk_tc,),
        in_specs=[
            pl.BlockSpec((1, SIM_ROWS, D_EMBED),
                         lambda j: ((off + j) // nrow, (off + j) % nrow, 0)),
            pl.BlockSpec((1, SEQ, D_EMBED), lambda j: ((off + j) // nrow, 0, 0)),
        ],
        out_specs=[
            pl.BlockSpec((SIM_ROWS, K), lambda j: (j, 0)),
            pl.BlockSpec((SIM_ROWS, K), lambda j: (j, 0)),
        ],
        out_shape=[
            jax.ShapeDtypeStruct((NROWS - S_SC, K), jnp.float32),
            jax.ShapeDtypeStruct((NROWS - S_SC, K), jnp.int32),
        ],
    )(embn3, embn3)

    sc_sc, sc_ix = pl.kernel(
        _sctopk_body,
        out_type=[jax.ShapeDtypeStruct((S_SC, K), jnp.float32),
                  jax.ShapeDtypeStruct((S_SC, K), jnp.int32)],
        mesh=plsc.VectorSubcoreMesh(core_axis_name="c", subcore_axis_name="s",
                                    num_cores=NC, num_subcores=NS),
        scratch_types=[
            pltpu.VMEM((RCHUNK, SEQ), jnp.float32),
            pltpu.VMEM((SEQ,), jnp.float32),
            pltpu.VMEM((SEQ,), jnp.int32),
            pltpu.VMEM((SEQ,), jnp.float32),
            pltpu.VMEM((SEQ,), jnp.int32),
            pltpu.VMEM((ROWS_PER_W, K), jnp.float32),
            pltpu.VMEM((ROWS_PER_W, K), jnp.int32),
        ],
        compiler_params=pltpu.CompilerParams(needs_layout_passes=False),
    )(sim)

    met, chrs, ecc = pl.pallas_call(
        _projc_body,
        grid=(nblk,),
        in_specs=[
            pl.BlockSpec((ROWS_PER_BLOCK, D_MODEL), lambda i: (i, 0)),
            _full((D_MODEL, D_MODEL)),
            _full((D_MODEL, N_CHR)), _full((1, N_CHR)),
            _full((D_MODEL, ECC_BITS)), _full((1, ECC_BITS)),
            _full((ECC_BITS, ECC_BITS * 2)), _full((1, ECC_BITS * 2)),
            _full((ECC_BITS * 2, ECC_BITS)), _full((1, ECC_BITS)),
        ],
        out_specs=[
            pl.BlockSpec((ROWS_PER_BLOCK, D_MODEL), lambda i: (i, 0)),
            pl.BlockSpec((ROWS_PER_BLOCK, N_CHR), lambda i: (i, 0)),
            pl.BlockSpec((ROWS_PER_BLOCK, ECC_BITS), lambda i: (i, 0)),
        ],
        out_shape=[
            jax.ShapeDtypeStruct((NROWS, D_MODEL), jnp.float32),
            jax.ShapeDtypeStruct((NROWS, N_CHR), jnp.float32),
            jax.ShapeDtypeStruct((NROWS, ECC_BITS), jnp.float32),
        ],
    )(xf, W_diag, W_chr, b_chr.reshape(1, -1),
      W_ecc_proj, b_ecc_proj.reshape(1, -1), W_e1, b_e1.reshape(1, -1),
      W_e2, b_e2.reshape(1, -1))

    metric = met.reshape(BATCH, SEQ, D_MODEL)
    christoffel = chrs.reshape(BATCH, SEQ, N_CHR)
    ecc_out = ecc.reshape(BATCH, SEQ, ECC_BITS)
    knn_scores = jnp.concatenate([sc_sc, tc_sc], axis=0).reshape(BATCH, SEQ, K)
    knn_indices = jnp.concatenate([sc_ix, tc_ix], axis=0).reshape(BATCH, SEQ, K)
    half_k = K // 2
    min_heap = knn_scores[..., :half_k]
    max_heap = -knn_scores[..., half_k:]
    return (embedding, metric, christoffel, knn_scores, knn_indices,
            min_heap, max_heap, ecc_out)


# speedup vs baseline: 1.8710x; 1.0612x over previous
import functools

import jax
import jax.numpy as jnp
from jax import lax
from jax.experimental import pallas as pl
from jax.experimental.pallas import tpu as pltpu
from jax.experimental.pallas import tpu_sc as plsc

D_MODEL = 1024
D_EMBED = 128
N_CHR = 32
ECC_BITS = 32
K = 32
BATCH = 4
SEQ = 2048
NROWS = BATCH * SEQ

ROWS_PER_BLOCK = 512
SIM_ROWS = 256

NC, NS = 2, 16
NW = NC * NS
S_SC = 4096
ROWS_PER_W = S_SC // NW
RCHUNK = 8
NCHUNKS = ROWS_PER_W // RCHUNK


def _proja_body(x_ref, we_ref, be_ref, emb_ref):
    c = D_MODEL // 4
    acc = jnp.dot(x_ref[:, :c], we_ref[:c, :], preferred_element_type=jnp.float32)
    for i in range(1, 4):
        acc = acc + jnp.dot(x_ref[:, i * c:(i + 1) * c], we_ref[i * c:(i + 1) * c, :],
                            preferred_element_type=jnp.float32)
    emb_ref[...] = acc + be_ref[...]


def _simmat_body(rows_ref, cols_ref, sim_ref):
    j = pl.program_id(0)
    sim = lax.dot_general(rows_ref[0], cols_ref[0], (((1,), (1,)), ((), ())),
                          preferred_element_type=jnp.float32)
    col_iota = lax.broadcasted_iota(jnp.int32, (SIM_ROWS, SEQ), 1)
    row_ids = ((j % (SEQ // SIM_ROWS)) * SIM_ROWS
               + lax.broadcasted_iota(jnp.int32, (SIM_ROWS, SEQ), 0))
    sim_ref[...] = jnp.where(col_iota == row_ids, sim - 1e9, sim)


def _tctopk_body(rows_ref, cols_ref, sc_ref, ix_ref):
    j = pl.program_id(0)
    sim = lax.dot_general(rows_ref[0], cols_ref[0], (((1,), (1,)), ((), ())),
                          preferred_element_type=jnp.float32)
    col_iota = lax.broadcasted_iota(jnp.int32, (SIM_ROWS, SEQ), 1)
    row_ids = (((S_SC // SIM_ROWS + j) % (SEQ // SIM_ROWS)) * SIM_ROWS
               + lax.broadcasted_iota(jnp.int32, (SIM_ROWS, SEQ), 0))
    sim = jnp.where(col_iota == row_ids, sim - 1e9, sim)
    scores = []
    idxs = []
    neg = jnp.float32(-jnp.inf)
    for _ in range(K):
        m = jnp.max(sim, axis=1, keepdims=True)
        am = jnp.min(jnp.where(sim == m, col_iota, SEQ), axis=1, keepdims=True)
        scores.append(m)
        idxs.append(am)
        sim = jnp.where(col_iota == am, neg, sim)
    sc_ref[...] = jnp.concatenate(scores, axis=1)
    ix_ref[...] = jnp.concatenate(idxs, axis=1)


def _projc_body(x_ref, wd_ref, wc_ref, bc_ref, wp_ref, bp_ref,
                w1_ref, b1_ref, w2_ref, b2_ref,
                met_ref, chr_ref, ecc_ref):
    x = x_ref[...]
    met_ref[...] = jnp.dot(x, wd_ref[...], preferred_element_type=jnp.float32)
    chr_ref[...] = jnp.dot(x, wc_ref[...], preferred_element_type=jnp.float32) + bc_ref[...]
    p = jnp.dot(x, wp_ref[...], preferred_element_type=jnp.float32) + bp_ref[...]
    h = jnp.tanh(jnp.dot(p, w1_ref[...], preferred_element_type=jnp.float32) + b1_ref[...])
    e = jnp.dot(h, w2_ref[...], preferred_element_type=jnp.float32) + b2_ref[...]
    ecc_ref[...] = 1.0 / (1.0 + jnp.exp(-e))


def _merge32(k1, v1, k2, v2, bk1, bv1, bk2, bv2):
    rbk2 = lax.rev(bk2, (0,)); rbv2 = lax.rev(bv2, (0,))
    rbk1 = lax.rev(bk1, (0,)); rbv1 = lax.rev(bv1, (0,))
    m1 = k1 >= rbk2
    t1k = jnp.where(m1, k1, rbk2); t1v = jnp.where(m1, v1, rbv2)
    m2 = k2 >= rbk1
    t2k = jnp.where(m2, k2, rbk1); t2v = jnp.where(m2, v2, rbv1)
    mm = t1k >= t2k
    u1k = jnp.where(mm, t1k, t2k); u1v = jnp.where(mm, t1v, t2v)
    u2k = jnp.where(mm, t2k, t1k); u2v = jnp.where(mm, t2v, t1v)
    s1k, s1v = plsc.sort_key_val(u1k, u1v, descending=True)
    s2k, s2v = plsc.sort_key_val(u2k, u2v, descending=True)
    return s1k, s1v, s2k, s2v


def _sctopk_body(sim_hbm, sc_hbm, ix_hbm,
                 rows0_v, rows1_v, key_v, idx_v, keyb_v, idxb_v, rs_v, ri_v,
                 sem0, sem1):
    wid = lax.axis_index("s") * NC + lax.axis_index("c")
    base = wid * ROWS_PER_W
    iota = lax.iota(jnp.int32, 16)
    bufs = (rows0_v, rows1_v)
    sems = (sem0, sem1)

    def _dma(g, b):
        return pltpu.make_async_copy(
            sim_hbm.at[pl.ds(base + g * RCHUNK, RCHUNK)], bufs[b], sems[b])

    _dma(0, 0).start()

    def chunk_pair(pair, _):
        for b in (0, 1):
            g = 2 * pair + b
            _dma(g, b).wait()

            @pl.when(g + 1 < NCHUNKS)
            def _():
                _dma(g + 1, 1 - b).start()

            rows_v = bufs[b]
            _process_chunk(g, rows_v, key_v, idx_v, keyb_v, idxb_v,
                           rs_v, ri_v, iota)
        return 0

    lax.fori_loop(0, NCHUNKS // 2, chunk_pair, 0)
    pltpu.sync_copy(rs_v, sc_hbm.at[pl.ds(base, ROWS_PER_W)])
    pltpu.sync_copy(ri_v, ix_hbm.at[pl.ds(base, ROWS_PER_W)])


def _process_chunk(g, rows_v, key_v, idx_v, keyb_v, idxb_v, rs_v, ri_v, iota):
        def row_body(r, _):
            def l01(j, _):
                off = j * 32
                ak = rows_v[r, pl.ds(off, 16)]
                bk = rows_v[r, pl.ds(off + 16, 16)]
                ia = off + iota
                ib = ia + 16
                sak, sav = plsc.sort_key_val(ak, ia, descending=True)
                sbk, sbv = plsc.sort_key_val(bk, ib, descending=True)
                rbk = lax.rev(sbk, (0,)); rbv = lax.rev(sbv, (0,))
                m = sak >= rbk
                u1k = jnp.where(m, sak, rbk); u1v = jnp.where(m, sav, rbv)
                u2k = jnp.where(m, rbk, sak); u2v = jnp.where(m, rbv, sav)
                s1k, s1v = plsc.sort_key_val(u1k, u1v, descending=True)
                s2k, s2v = plsc.sort_key_val(u2k, u2v, descending=True)
                key_v[pl.ds(off, 16)] = s1k
                key_v[pl.ds(off + 16, 16)] = s2k
                idx_v[pl.ds(off, 16)] = s1v
                idx_v[pl.ds(off + 16, 16)] = s2v
                return 0

            lax.fori_loop(0, SEQ // 32, l01, 0, unroll=2)

            nlists = SEQ // 32
            srck, srci, dstk, dsti = key_v, idx_v, keyb_v, idxb_v
            while nlists > 1:
                def lvl(j, _, srck=srck, srci=srci, dstk=dstk, dsti=dsti):
                    a = j * 64
                    r_ = _merge32(
                        srck[pl.ds(a, 16)], srci[pl.ds(a, 16)],
                        srck[pl.ds(a + 16, 16)], srci[pl.ds(a + 16, 16)],
                        srck[pl.ds(a + 32, 16)], srci[pl.ds(a + 32, 16)],
                        srck[pl.ds(a + 48, 16)], srci[pl.ds(a + 48, 16)])
                    o = j * 32
                    dstk[pl.ds(o, 16)] = r_[0]
                    dstk[pl.ds(o + 16, 16)] = r_[2]
                    dsti[pl.ds(o, 16)] = r_[1]
                    dsti[pl.ds(o + 16, 16)] = r_[3]
                    return 0

                lax.fori_loop(0, nlists // 2, lvl, 0,
                              unroll=2 if nlists > 4 else 1)
                srck, srci, dstk, dsti = dstk, dsti, srck, srci
                nlists //= 2

            rr = g * RCHUNK + r
            rs_v[rr, pl.ds(0, 16)] = srck[pl.ds(0, 16)]
            rs_v[rr, pl.ds(16, 16)] = srck[pl.ds(16, 16)]
            ri_v[rr, pl.ds(0, 16)] = srci[pl.ds(0, 16)]
            ri_v[rr, pl.ds(16, 16)] = srci[pl.ds(16, 16)]
            return 0

        lax.fori_loop(0, RCHUNK, row_body, 0)


def kernel(x, W_embed, b_embed, W_diag, W_chr, b_chr,
           W_ecc_proj, b_ecc_proj, W_e1, b_e1, W_e2, b_e2):
    xf = x.reshape(NROWS, D_MODEL)
    nblk = NROWS // ROWS_PER_BLOCK

    def _full(shape):
        return pl.BlockSpec(shape, lambda i: (0,) * len(shape))

    emb = pl.pallas_call(
        _proja_body,
        grid=(nblk,),
        in_specs=[
            pl.BlockSpec((ROWS_PER_BLOCK, D_MODEL), lambda i: (i, 0)),
            _full((D_MODEL, D_EMBED)), _full((1, D_EMBED)),
        ],
        out_specs=pl.BlockSpec((ROWS_PER_BLOCK, D_EMBED), lambda i: (i, 0)),
        out_shape=jax.ShapeDtypeStruct((NROWS, D_EMBED), jnp.float32),
    )(xf, W_embed, b_embed.reshape(1, -1))

    embedding = emb.reshape(BATCH, SEQ, D_EMBED)
    embn3 = embedding / (jnp.linalg.norm(embedding, axis=-1, keepdims=True) + 1e-8)
    nrow = SEQ // SIM_ROWS
    nblk_sc = S_SC // SIM_ROWS
    sim = pl.pallas_call(
        _simmat_body,
        grid=(nblk_sc,),
        in_specs=[
            pl.BlockSpec((1, SIM_ROWS, D_EMBED),
                         lambda j: (j // nrow, j % nrow, 0)),
            pl.BlockSpec((1, SEQ, D_EMBED), lambda j: (j // nrow, 0, 0)),
        ],
        out_specs=pl.BlockSpec((SIM_ROWS, SEQ), lambda j: (j, 0)),
        out_shape=jax.ShapeDtypeStruct((S_SC, SEQ), jnp.float32),
    )(embn3, embn3)

    nblk_tc = (NROWS - S_SC) // SIM_ROWS
    off = S_SC // SIM_ROWS
    tc_sc, tc_ix = pl.pallas_call(
        _tctopk_body,
        grid=(nblk_tc,),
        in_specs=[
            pl.BlockSpec((1, SIM_ROWS, D_EMBED),
                         lambda j: ((off + j) // nrow, (off + j) % nrow, 0)),
            pl.BlockSpec((1, SEQ, D_EMBED), lambda j: ((off + j) // nrow, 0, 0)),
        ],
        out_specs=[
            pl.BlockSpec((SIM_ROWS, K), lambda j: (j, 0)),
            pl.BlockSpec((SIM_ROWS, K), lambda j: (j, 0)),
        ],
        out_shape=[
            jax.ShapeDtypeStruct((NROWS - S_SC, K), jnp.float32),
            jax.ShapeDtypeStruct((NROWS - S_SC, K), jnp.int32),
        ],
    )(embn3, embn3)

    sc_sc, sc_ix = pl.kernel(
        _sctopk_body,
        out_type=[jax.ShapeDtypeStruct((S_SC, K), jnp.float32),
                  jax.ShapeDtypeStruct((S_SC, K), jnp.int32)],
        mesh=plsc.VectorSubcoreMesh(core_axis_name="c", subcore_axis_name="s",
                                    num_cores=NC, num_subcores=NS),
        scratch_types=[
            pltpu.VMEM((RCHUNK, SEQ), jnp.float32),
            pltpu.VMEM((RCHUNK, SEQ), jnp.float32),
            pltpu.VMEM((SEQ,), jnp.float32),
            pltpu.VMEM((SEQ,), jnp.int32),
            pltpu.VMEM((SEQ,), jnp.float32),
            pltpu.VMEM((SEQ,), jnp.int32),
            pltpu.VMEM((ROWS_PER_W, K), jnp.float32),
            pltpu.VMEM((ROWS_PER_W, K), jnp.int32),
            pltpu.SemaphoreType.DMA,
            pltpu.SemaphoreType.DMA,
        ],
        compiler_params=pltpu.CompilerParams(needs_layout_passes=False),
    )(sim)

    met, chrs, ecc = pl.pallas_call(
        _projc_body,
        grid=(nblk,),
        in_specs=[
            pl.BlockSpec((ROWS_PER_BLOCK, D_MODEL), lambda i: (i, 0)),
            _full((D_MODEL, D_MODEL)),
            _full((D_MODEL, N_CHR)), _full((1, N_CHR)),
            _full((D_MODEL, ECC_BITS)), _full((1, ECC_BITS)),
            _full((ECC_BITS, ECC_BITS * 2)), _full((1, ECC_BITS * 2)),
            _full((ECC_BITS * 2, ECC_BITS)), _full((1, ECC_BITS)),
        ],
        out_specs=[
            pl.BlockSpec((ROWS_PER_BLOCK, D_MODEL), lambda i: (i, 0)),
            pl.BlockSpec((ROWS_PER_BLOCK, N_CHR), lambda i: (i, 0)),
            pl.BlockSpec((ROWS_PER_BLOCK, ECC_BITS), lambda i: (i, 0)),
        ],
        out_shape=[
            jax.ShapeDtypeStruct((NROWS, D_MODEL), jnp.float32),
            jax.ShapeDtypeStruct((NROWS, N_CHR), jnp.float32),
            jax.ShapeDtypeStruct((NROWS, ECC_BITS), jnp.float32),
        ],
    )(xf, W_diag, W_chr, b_chr.reshape(1, -1),
      W_ecc_proj, b_ecc_proj.reshape(1, -1), W_e1, b_e1.reshape(1, -1),
      W_e2, b_e2.reshape(1, -1))

    metric = met.reshape(BATCH, SEQ, D_MODEL)
    christoffel = chrs.reshape(BATCH, SEQ, N_CHR)
    ecc_out = ecc.reshape(BATCH, SEQ, ECC_BITS)
    knn_scores = jnp.concatenate([sc_sc, tc_sc], axis=0).reshape(BATCH, SEQ, K)
    knn_indices = jnp.concatenate([sc_ix, tc_ix], axis=0).reshape(BATCH, SEQ, K)
    half_k = K // 2
    min_heap = knn_scores[..., :half_k]
    max_heap = -knn_scores[..., half_k:]
    return (embedding, metric, christoffel, knn_scores, knn_indices,
            min_heap, max_heap, ecc_out)
